# Initial kernel scaffold; baseline (speedup 1.0000x reference)
import jax
import jax.numpy as jnp
from jax.experimental import pallas as pl


def kernel(x, edge_index, W, a):
    raise NotImplementedError("write your pallas kernel here")


# speedup vs baseline: 16.4233x; 16.4233x over previous
import functools

import jax
import jax.numpy as jnp
from jax import lax
from jax.experimental import pallas as pl
from jax.experimental.pallas import tpu as pltpu
from jax.experimental.pallas import tpu_sc as plsc

N = 10000
E = 320000
D = 128
DH = D // 2
ALPHA = 0.2

NC = 2
NS = 16
L = 16
NW = NC * NS
EPW = E // NW
EPT = E // NS
CH = 80
NCH_D = EPW // CH
NCH_A = EPT // CH
NPAD = 10240
RPT = NPAD // NS

_mesh = plsc.VectorSubcoreMesh(core_axis_name="c", subcore_axis_name="s")
_sc_params = pltpu.CompilerParams(
    needs_layout_passes=False, use_tc_tiling_on_sc=False)


def _proj_body(x_ref, w0_ref, w1_ref, a80_ref, a81_ref, h2_ref, s8_ref):
    x = x_ref[...]
    h0 = jnp.dot(x, w0_ref[...], preferred_element_type=jnp.float32)
    h1 = jnp.dot(x, w1_ref[...], preferred_element_type=jnp.float32)
    h2_ref[0] = h0
    h2_ref[1] = h1
    s8_ref[...] = (
        lax.dot_general(a80_ref[...], h0, (((1,), (1,)), ((), ())),
                        preferred_element_type=jnp.float32)
        + lax.dot_general(a81_ref[...], h1, (((1,), (1,)), ((), ())),
                          preferred_element_type=jnp.float32))


_proj = pl.pallas_call(
    _proj_body,
    out_shape=[
        jax.ShapeDtypeStruct((NC, N, DH), jnp.float32),
        jax.ShapeDtypeStruct((8, N), jnp.float32),
    ],
)


def _denom_body(s8, src2, dst2, zn, dpart, s1_v, s2_v, si_v, di_v, w_v, dacc):
    c = lax.axis_index("c")
    s = lax.axis_index("s")
    wid = c * NS + s
    pltpu.sync_copy(s8.at[0], s1_v)
    pltpu.sync_copy(s8.at[1], s2_v)
    pltpu.sync_copy(src2.at[wid], si_v)
    pltpu.sync_copy(dst2.at[wid], di_v)
    pltpu.sync_copy(zn.at[pl.ds(s * RPT, RPT)], dacc.at[pl.ds(s * RPT, RPT)])

    @pl.loop(0, NCH_D)
    def _compute(j):
        for k in range(CH // L):
            si = si_v[j, pl.ds(k * L, L)]
            di = di_v[j, pl.ds(k * L, L)]
            e = plsc.load_gather(s1_v, [si]) + plsc.load_gather(s2_v, [di])
            e = jnp.where(e > 0.0, e, ALPHA * e)
            w_v[j, pl.ds(k * L, L)] = jnp.exp(e)

    plsc.subcore_barrier()

    @pl.loop(0, NCH_D)
    def _scatter(j):
        pltpu.sync_copy(w_v.at[j], dacc.at[si_v.at[j]], add=True)

    plsc.subcore_barrier()
    pltpu.sync_copy(dacc.at[pl.ds(s * RPT, RPT)],
                    dpart.at[c, pl.ds(s * RPT, RPT)])


_denom = functools.partial(
    pl.kernel,
    out_type=jax.ShapeDtypeStruct((NC, NPAD), jnp.float32),
    mesh=_mesh,
    scratch_types=[
        pltpu.VMEM((N,), jnp.float32),
        pltpu.VMEM((N,), jnp.float32),
        pltpu.VMEM((NCH_D, CH), jnp.int32),
        pltpu.VMEM((NCH_D, CH), jnp.int32),
        pltpu.VMEM((NCH_D, CH), jnp.float32),
        pltpu.VMEM_SHARED((NPAD,), jnp.float32),
    ],
    compiler_params=_sc_params,
)(_denom_body)


def _agg_body(h2, s8, src2, dst2, dpart, zr, out,
              s1_v, s2_v, d_v, d1_v, si_v, di_v, rows_v, acc):
    c = lax.axis_index("c")
    s = lax.axis_index("s")
    pltpu.sync_copy(s8.at[0], s1_v)
    pltpu.sync_copy(s8.at[1], s2_v)
    pltpu.sync_copy(dpart.at[0], d_v)
    pltpu.sync_copy(dpart.at[1], d1_v)
    pltpu.sync_copy(src2.at[s], si_v)
    pltpu.sync_copy(dst2.at[s], di_v)
    pltpu.sync_copy(zr.at[pl.ds(s * RPT, RPT)], acc.at[pl.ds(s * RPT, RPT)])

    @pl.loop(0, NPAD // L)
    def _sum_denoms(i):
        d_v[pl.ds(i * L, L)] = d_v[pl.ds(i * L, L)] + d1_v[pl.ds(i * L, L)]

    plsc.subcore_barrier()

    @pl.loop(0, NCH_A)
    def _chunk(j):
        pltpu.sync_copy(h2.at[c].at[si_v.at[j]], rows_v)
        for k in range(CH // L):
            si = si_v[j, pl.ds(k * L, L)]
            di = di_v[j, pl.ds(k * L, L)]
            e = plsc.load_gather(s1_v, [si]) + plsc.load_gather(s2_v, [di])
            e = jnp.where(e > 0.0, e, ALPHA * e)
            att16 = jnp.exp(e) / plsc.load_gather(d_v, [si])
            for t in range(L):
                a = att16[t]
                r = k * L + t
                for cb in range(DH // L):
                    rows_v[r, pl.ds(cb * L, L)] = rows_v[r, pl.ds(cb * L, L)] * a

        pltpu.sync_copy(rows_v, acc.at[di_v.at[j]], add=True)

    plsc.subcore_barrier()
    pltpu.sync_copy(acc.at[pl.ds(s * RPT, RPT)],
                    out.at[c, pl.ds(s * RPT, RPT)])


_agg = functools.partial(
    pl.kernel,
    out_type=jax.ShapeDtypeStruct((NC, NPAD, DH), jnp.float32),
    mesh=_mesh,
    scratch_types=[
        pltpu.VMEM((N,), jnp.float32),
        pltpu.VMEM((N,), jnp.float32),
        pltpu.VMEM((NPAD,), jnp.float32),
        pltpu.VMEM((NPAD,), jnp.float32),
        pltpu.VMEM((NCH_A, CH), jnp.int32),
        pltpu.VMEM((NCH_A, CH), jnp.int32),
        pltpu.VMEM((CH, DH), jnp.float32),
        pltpu.VMEM_SHARED((NPAD, DH), jnp.float32),
    ],
    compiler_params=_sc_params,
)(_agg_body)


_BR3 = 640


def _elu_body(acc_ref, o_ref):
    y0 = acc_ref[0]
    y1 = acc_ref[1]
    o_ref[:, :DH] = jnp.where(y0 > 0.0, y0, jnp.exp(y0) - 1.0)
    o_ref[:, DH:] = jnp.where(y1 > 0.0, y1, jnp.exp(y1) - 1.0)


_elu = pl.pallas_call(
    _elu_body,
    grid=(NPAD // _BR3,),
    in_specs=[pl.BlockSpec((NC, _BR3, DH), lambda i: (0, i, 0))],
    out_specs=pl.BlockSpec((_BR3, D), lambda i: (i, 0)),
    out_shape=jax.ShapeDtypeStruct((NPAD, D), jnp.float32),
)


def kernel(x, edge_index, W, a):
    a1 = a[:D]
    a2 = a[D:]
    a80 = jnp.zeros((8, DH), jnp.float32).at[0].set(a1[:DH]).at[1].set(a2[:DH])
    a81 = jnp.zeros((8, DH), jnp.float32).at[0].set(a1[DH:]).at[1].set(a2[DH:])
    h2, s8 = _proj(x, W[:, :DH], W[:, DH:], a80, a81)
    src_d = edge_index[0].reshape(NW, NCH_D, CH)
    dst_d = edge_index[1].reshape(NW, NCH_D, CH)
    src_a = edge_index[0].reshape(NS, NCH_A, CH)
    dst_a = edge_index[1].reshape(NS, NCH_A, CH)
    zn = jnp.zeros((NPAD,), jnp.float32)
    zr = jnp.zeros((NPAD, DH), jnp.float32)
    dpart = _denom(s8, src_d, dst_d, zn)
    accs = _agg(h2, s8, src_a, dst_a, dpart, zr)
    return _elu(accs)[:N]


# speedup vs baseline: 24.8081x; 1.5105x over previous
import functools

import jax
import jax.numpy as jnp
from jax import lax
from jax.experimental import pallas as pl
from jax.experimental.pallas import tpu as pltpu
from jax.experimental.pallas import tpu_sc as plsc

N = 10000
E = 320000
D = 128
DH = D // 2
ALPHA = 0.2

NC = 2
NS = 16
L = 16
NW = NC * NS
EPW = E // NW
EPT = E // NS
CH = 80
NCH_D = EPW // CH
NCH_A = EPT // CH
NPAD = 10240
RPT = NPAD // NS

_mesh = plsc.VectorSubcoreMesh(core_axis_name="c", subcore_axis_name="s")
_sc_params = pltpu.CompilerParams(
    needs_layout_passes=False, use_tc_tiling_on_sc=False)


def _proj_body(x_ref, w0_ref, w1_ref, a80_ref, a81_ref, h2_ref, s8_ref):
    x = x_ref[...]
    h0 = jnp.dot(x, w0_ref[...], preferred_element_type=jnp.float32)
    h1 = jnp.dot(x, w1_ref[...], preferred_element_type=jnp.float32)
    h2_ref[0] = h0
    h2_ref[1] = h1
    s8_ref[...] = (
        lax.dot_general(a80_ref[...], h0, (((1,), (1,)), ((), ())),
                        preferred_element_type=jnp.float32)
        + lax.dot_general(a81_ref[...], h1, (((1,), (1,)), ((), ())),
                          preferred_element_type=jnp.float32))


_proj = pl.pallas_call(
    _proj_body,
    out_shape=[
        jax.ShapeDtypeStruct((NC, N, DH), jnp.float32),
        jax.ShapeDtypeStruct((8, N), jnp.float32),
    ],
)


def _denom_body(s8, src2, dst2, zn, dpart, w_hbm,
                s1_v, s2_v, si_v, di_v, w_v, dacc, dsem):
    c = lax.axis_index("c")
    s = lax.axis_index("s")
    wid = c * NS + s
    pltpu.sync_copy(s8.at[0], s1_v)
    pltpu.sync_copy(s8.at[1], s2_v)
    pltpu.sync_copy(src2.at[wid], si_v)
    pltpu.sync_copy(dst2.at[wid], di_v)
    pltpu.sync_copy(zn.at[pl.ds(s * RPT, RPT)], dacc.at[pl.ds(s * RPT, RPT)])

    @pl.loop(0, NCH_D)
    def _compute(j):
        for k in range(CH // L):
            si = si_v[j, pl.ds(k * L, L)]
            di = di_v[j, pl.ds(k * L, L)]
            e = plsc.load_gather(s1_v, [si]) + plsc.load_gather(s2_v, [di])
            e = jnp.where(e > 0.0, e, ALPHA * e)
            w_v[j, pl.ds(k * L, L)] = jnp.exp(e)

    pltpu.sync_copy(w_v, w_hbm.at[wid])
    plsc.subcore_barrier()

    @pl.loop(0, NCH_D)
    def _scatter(j):
        pltpu.async_copy(w_v.at[j], dacc.at[si_v.at[j]], dsem, add=True)

        @pl.when(j >= 8)
        def _throttle():
            pltpu.make_async_copy(w_v.at[0], dacc.at[si_v.at[0]], dsem).wait()

    @pl.loop(0, 8)
    def _drain(j):
        pltpu.make_async_copy(w_v.at[0], dacc.at[si_v.at[0]], dsem).wait()

    plsc.subcore_barrier()
    pltpu.sync_copy(dacc.at[pl.ds(s * RPT, RPT)],
                    dpart.at[c, pl.ds(s * RPT, RPT)])


_denom = functools.partial(
    pl.kernel,
    out_type=[
        jax.ShapeDtypeStruct((NC, NPAD), jnp.float32),
        jax.ShapeDtypeStruct((NW, NCH_D, CH), jnp.float32),
    ],
    mesh=_mesh,
    scratch_types=[
        pltpu.VMEM((N,), jnp.float32),
        pltpu.VMEM((N,), jnp.float32),
        pltpu.VMEM((NCH_D, CH), jnp.int32),
        pltpu.VMEM((NCH_D, CH), jnp.int32),
        pltpu.VMEM((NCH_D, CH), jnp.float32),
        pltpu.VMEM_SHARED((NPAD,), jnp.float32),
        pltpu.SemaphoreType.DMA,
    ],
    compiler_params=_sc_params,
)(_denom_body)


NB = 5
NQ = NCH_A // NB


def _agg_body(h2, w3, src2, dst2, dpart, out,
              d_v, d1_v, si_v, di_v,
              r0, r1, r2, r3, r4, w0, w1, w2, w3_, w4,
              g0, g1, g2, g3, g4, t0, t1, t2, t3, t4, acc):
    rows = (r0, r1, r2, r3, r4)
    wring = (w0, w1, w2, w3_, w4)
    gsem = (g0, g1, g2, g3, g4)
    ssem = (t0, t1, t2, t3, t4)
    c = lax.axis_index("c")
    s = lax.axis_index("s")
    pltpu.sync_copy(dpart.at[0], d_v)
    pltpu.sync_copy(dpart.at[1], d1_v)
    pltpu.sync_copy(src2.at[s], si_v)
    pltpu.sync_copy(dst2.at[s], di_v)

    @pl.loop(0, CH)
    def _zrow(r):
        for cb in range(DH // L):
            r0[r, pl.ds(cb * L, L)] = jnp.zeros((L,), jnp.float32)

    for p in range(RPT // CH):
        pltpu.sync_copy(r0, acc.at[pl.ds(s * RPT + p * CH, CH)])

    @pl.loop(0, NPAD // L)
    def _sum_denoms(i):
        d_v[pl.ds(i * L, L)] = d_v[pl.ds(i * L, L)] + d1_v[pl.ds(i * L, L)]

    def _gather(j, b):
        pltpu.async_copy(h2.at[c].at[si_v.at[j]], rows[b], gsem[b])
        pltpu.async_copy(w3.at[s].at[j], wring[b], gsem[b])

    def _wait_gather(b):
        pltpu.make_async_copy(h2.at[c].at[si_v.at[0]], rows[b], gsem[b]).wait()
        pltpu.make_async_copy(w3.at[s].at[0], wring[b], gsem[b]).wait()

    def _scatter(j, b):
        pltpu.async_copy(rows[b], acc.at[di_v.at[j]], ssem[b], add=True)

    def _wait_scatter(b):
        pltpu.make_async_copy(rows[b], acc.at[di_v.at[0]], ssem[b]).wait()

    def _compute(j, b):
        rv = rows[b]
        wv = wring[b]
        for k in range(CH // L):
            si = si_v[j, pl.ds(k * L, L)]
            att16 = wv[pl.ds(k * L, L)] / plsc.load_gather(d_v, [si])
            for t in range(L):
                a = att16[t]
                r = k * L + t
                for cb in range(DH // L):
                    rv[r, pl.ds(cb * L, L)] = rv[r, pl.ds(cb * L, L)] * a

    plsc.subcore_barrier()

    _gather(0, 0)
    _gather(1, 1)

    @pl.loop(0, NQ)
    def _pipe(q):
        for i in range(NB):
            j = q * NB + i
            b2 = (i + 2) % NB
            jn = j + 2
            _wait_gather(i)
            _compute(j, i)
            _scatter(j, i)

            @pl.when(jnp.logical_and(jn >= NB, jn < NCH_A))
            def _():
                _wait_scatter(b2)

            @pl.when(jn < NCH_A)
            def _():
                _gather(jn, b2)

    for b in range(NB):
        _wait_scatter(b)

    plsc.subcore_barrier()
    pltpu.sync_copy(acc.at[pl.ds(s * RPT, RPT)],
                    out.at[c, pl.ds(s * RPT, RPT)])


_agg = functools.partial(
    pl.kernel,
    out_type=jax.ShapeDtypeStruct((NC, NPAD, DH), jnp.float32),
    mesh=_mesh,
    scratch_types=[
        pltpu.VMEM((NPAD,), jnp.float32),
        pltpu.VMEM((NPAD,), jnp.float32),
        pltpu.VMEM((NCH_A, CH), jnp.int32),
        pltpu.VMEM((NCH_A, CH), jnp.int32),
    ] + [pltpu.VMEM((CH, DH), jnp.float32) for _ in range(NB)]
    + [pltpu.VMEM((CH,), jnp.float32) for _ in range(NB)]
    + [pltpu.SemaphoreType.DMA for _ in range(2 * NB)]
    + [
        pltpu.VMEM_SHARED((NPAD, DH), jnp.float32),
    ],
    compiler_params=_sc_params,
)(_agg_body)


_BR3 = 640


def _elu_body(acc_ref, o_ref):
    y0 = acc_ref[0]
    y1 = acc_ref[1]
    o_ref[:, :DH] = jnp.where(y0 > 0.0, y0, jnp.exp(y0) - 1.0)
    o_ref[:, DH:] = jnp.where(y1 > 0.0, y1, jnp.exp(y1) - 1.0)


_elu = pl.pallas_call(
    _elu_body,
    grid=(NPAD // _BR3,),
    in_specs=[pl.BlockSpec((NC, _BR3, DH), lambda i: (0, i, 0))],
    out_specs=pl.BlockSpec((_BR3, D), lambda i: (i, 0)),
    out_shape=jax.ShapeDtypeStruct((NPAD, D), jnp.float32),
)


def kernel(x, edge_index, W, a):
    a1 = a[:D]
    a2 = a[D:]
    a80 = jnp.zeros((8, DH), jnp.float32).at[0].set(a1[:DH]).at[1].set(a2[:DH])
    a81 = jnp.zeros((8, DH), jnp.float32).at[0].set(a1[DH:]).at[1].set(a2[DH:])
    h2, s8 = _proj(x, W[:, :DH], W[:, DH:], a80, a81)
    src_d = edge_index[0].reshape(NW, NCH_D, CH)
    dst_d = edge_index[1].reshape(NW, NCH_D, CH)
    src_a = edge_index[0].reshape(NS, NCH_A, CH)
    dst_a = edge_index[1].reshape(NS, NCH_A, CH)
    zn = jnp.zeros((NPAD,), jnp.float32)
    dpart, w = _denom(s8, src_d, dst_d, zn)
    w_a = w.reshape(NS, NCH_A, CH)
    accs = _agg(h2, w_a, src_a, dst_a, dpart)
    return _elu(accs)[:N]


# speedup vs baseline: 28.8186x; 1.1617x over previous
import functools

import jax
import jax.numpy as jnp
from jax import lax
from jax.experimental import pallas as pl
from jax.experimental.pallas import tpu as pltpu
from jax.experimental.pallas import tpu_sc as plsc

N = 10000
E = 320000
D = 128
DH = D // 2
ALPHA = 0.2

NC = 2
NS = 16
L = 16
NW = NC * NS
EPW = E // NW
EPT = E // NS
CH = 80
NCH_D = EPW // CH
NCH_A = EPT // CH
NPAD = 10240
RPT = NPAD // NS

_mesh = plsc.VectorSubcoreMesh(core_axis_name="c", subcore_axis_name="s")
_sc_params = pltpu.CompilerParams(
    needs_layout_passes=False, use_tc_tiling_on_sc=False)


def _proj_body(x_ref, w0_ref, w1_ref, a80_ref, a81_ref, h2_ref, s8_ref):
    x = x_ref[...]
    h0 = jnp.dot(x, w0_ref[...], preferred_element_type=jnp.float32)
    h1 = jnp.dot(x, w1_ref[...], preferred_element_type=jnp.float32)
    h2_ref[0] = h0
    h2_ref[1] = h1
    s8_ref[...] = (
        lax.dot_general(a80_ref[...], h0, (((1,), (1,)), ((), ())),
                        preferred_element_type=jnp.float32)
        + lax.dot_general(a81_ref[...], h1, (((1,), (1,)), ((), ())),
                          preferred_element_type=jnp.float32))


_proj = pl.pallas_call(
    _proj_body,
    out_shape=[
        jax.ShapeDtypeStruct((NC, N, DH), jnp.float32),
        jax.ShapeDtypeStruct((8, N), jnp.float32),
    ],
)


def _denom_body(s8, src2, dst2, zn, dpart, w_hbm,
                s1_v, s2_v, si_v, di_v, w_v, dacc, dsem):
    c = lax.axis_index("c")
    s = lax.axis_index("s")
    wid = c * NS + s
    pltpu.sync_copy(s8.at[0], s1_v)
    pltpu.sync_copy(s8.at[1], s2_v)
    pltpu.sync_copy(src2.at[wid], si_v)
    pltpu.sync_copy(dst2.at[wid], di_v)
    pltpu.sync_copy(zn.at[pl.ds(s * RPT, RPT)], dacc.at[pl.ds(s * RPT, RPT)])

    @pl.loop(0, NCH_D)
    def _compute(j):
        for k in range(CH // L):
            si = si_v[j, pl.ds(k * L, L)]
            di = di_v[j, pl.ds(k * L, L)]
            e = plsc.load_gather(s1_v, [si]) + plsc.load_gather(s2_v, [di])
            e = jnp.where(e > 0.0, e, ALPHA * e)
            w_v[j, pl.ds(k * L, L)] = jnp.exp(e)

    pltpu.sync_copy(w_v, w_hbm.at[wid])
    plsc.subcore_barrier()

    @pl.loop(0, NCH_D)
    def _scatter(j):
        pltpu.async_copy(w_v.at[j], dacc.at[si_v.at[j]], dsem, add=True)

        @pl.when(j >= 8)
        def _throttle():
            pltpu.make_async_copy(w_v.at[0], dacc.at[si_v.at[0]], dsem).wait()

    @pl.loop(0, 8)
    def _drain(j):
        pltpu.make_async_copy(w_v.at[0], dacc.at[si_v.at[0]], dsem).wait()

    plsc.subcore_barrier()
    pltpu.sync_copy(dacc.at[pl.ds(s * RPT, RPT)],
                    dpart.at[c, pl.ds(s * RPT, RPT)])


_denom = functools.partial(
    pl.kernel,
    out_type=[
        jax.ShapeDtypeStruct((NC, NPAD), jnp.float32),
        jax.ShapeDtypeStruct((NW, NCH_D, CH), jnp.float32),
    ],
    mesh=_mesh,
    scratch_types=[
        pltpu.VMEM((N,), jnp.float32),
        pltpu.VMEM((N,), jnp.float32),
        pltpu.VMEM((NCH_D, CH), jnp.int32),
        pltpu.VMEM((NCH_D, CH), jnp.int32),
        pltpu.VMEM((NCH_D, CH), jnp.float32),
        pltpu.VMEM_SHARED((NPAD,), jnp.float32),
        pltpu.SemaphoreType.DMA,
    ],
    compiler_params=_sc_params,
)(_denom_body)


NB = 5
NQ = NCH_A // NB


def _agg_body(h2, w3, src2, dst2, dpart, out,
              d_v, d1_v, si_v, di_v,
              r0, r1, r2, r3, r4, w0, w1, w2, w3_, w4,
              g0, g1, g2, g3, g4, t0, t1, t2, t3, t4, acc):
    rows = (r0, r1, r2, r3, r4)
    wring = (w0, w1, w2, w3_, w4)
    gsem = (g0, g1, g2, g3, g4)
    ssem = (t0, t1, t2, t3, t4)
    c = lax.axis_index("c")
    s = lax.axis_index("s")
    pltpu.sync_copy(dpart.at[0], d_v)
    pltpu.sync_copy(dpart.at[1], d1_v)
    pltpu.sync_copy(src2.at[s], si_v)
    pltpu.sync_copy(dst2.at[s], di_v)

    @pl.loop(0, CH)
    def _zrow(r):
        for cb in range(DH // L):
            r0[r, pl.ds(cb * L, L)] = jnp.zeros((L,), jnp.float32)

    for p in range(RPT // CH):
        pltpu.sync_copy(r0, acc.at[pl.ds(s * RPT + p * CH, CH)])

    @pl.loop(0, NPAD // L)
    def _sum_denoms(i):
        d_v[pl.ds(i * L, L)] = d_v[pl.ds(i * L, L)] + d1_v[pl.ds(i * L, L)]

    def _gather(j, b):
        pltpu.async_copy(h2.at[c].at[si_v.at[j]], rows[b], gsem[b])
        pltpu.async_copy(w3.at[s].at[j], wring[b], gsem[b])

    def _wait_gather(b):
        pltpu.make_async_copy(h2.at[c].at[si_v.at[0]], rows[b], gsem[b]).wait()
        pltpu.make_async_copy(w3.at[s].at[0], wring[b], gsem[b]).wait()

    def _scatter(j, b):
        pltpu.async_copy(rows[b], acc.at[di_v.at[j]], ssem[b], add=True)

    def _wait_scatter(b):
        pltpu.make_async_copy(rows[b], acc.at[di_v.at[0]], ssem[b]).wait()

    def _compute(j, b):
        rv = rows[b]
        wv = wring[b]
        for k in range(CH // L):
            si = si_v[j, pl.ds(k * L, L)]
            att16 = wv[pl.ds(k * L, L)] / plsc.load_gather(d_v, [si])
            for t in range(L):
                a = att16[t]
                r = k * L + t
                for cb in range(DH // L):
                    rv[r, pl.ds(cb * L, L)] = rv[r, pl.ds(cb * L, L)] * a

    plsc.subcore_barrier()

    _gather(0, 0)
    _gather(1, 1)
    _gather(2, 2)

    @pl.loop(0, NQ)
    def _pipe(q):
        for i in range(NB):
            j = q * NB + i
            b3 = (i + 3) % NB
            jn = j + 3
            _wait_gather(i)

            @pl.when(jnp.logical_and(jn >= NB, jn < NCH_A))
            def _():
                _wait_scatter(b3)

            @pl.when(jn < NCH_A)
            def _():
                _gather(jn, b3)

            _compute(j, i)
            _scatter(j, i)

    for b in range(NB):
        _wait_scatter(b)

    plsc.subcore_barrier()
    pltpu.sync_copy(acc.at[pl.ds(s * RPT, RPT)],
                    out.at[c, pl.ds(s * RPT, RPT)])


_agg = functools.partial(
    pl.kernel,
    out_type=jax.ShapeDtypeStruct((NC, NPAD, DH), jnp.float32),
    mesh=_mesh,
    scratch_types=[
        pltpu.VMEM((NPAD,), jnp.float32),
        pltpu.VMEM((NPAD,), jnp.float32),
        pltpu.VMEM((NCH_A, CH), jnp.int32),
        pltpu.VMEM((NCH_A, CH), jnp.int32),
    ] + [pltpu.VMEM((CH, DH), jnp.float32) for _ in range(NB)]
    + [pltpu.VMEM((CH,), jnp.float32) for _ in range(NB)]
    + [pltpu.SemaphoreType.DMA for _ in range(2 * NB)]
    + [
        pltpu.VMEM_SHARED((NPAD, DH), jnp.float32),
    ],
    compiler_params=_sc_params,
)(_agg_body)


_BR3 = 640


def _elu_body(acc_ref, o_ref):
    y0 = acc_ref[0]
    y1 = acc_ref[1]
    o_ref[:, :DH] = jnp.where(y0 > 0.0, y0, jnp.exp(y0) - 1.0)
    o_ref[:, DH:] = jnp.where(y1 > 0.0, y1, jnp.exp(y1) - 1.0)


_elu = pl.pallas_call(
    _elu_body,
    grid=(NPAD // _BR3,),
    in_specs=[pl.BlockSpec((NC, _BR3, DH), lambda i: (0, i, 0))],
    out_specs=pl.BlockSpec((_BR3, D), lambda i: (i, 0)),
    out_shape=jax.ShapeDtypeStruct((NPAD, D), jnp.float32),
)


def kernel(x, edge_index, W, a):
    a1 = a[:D]
    a2 = a[D:]
    a80 = jnp.zeros((8, DH), jnp.float32).at[0].set(a1[:DH]).at[1].set(a2[:DH])
    a81 = jnp.zeros((8, DH), jnp.float32).at[0].set(a1[DH:]).at[1].set(a2[DH:])
    h2, s8 = _proj(x, W[:, :DH], W[:, DH:], a80, a81)
    src_d = edge_index[0].reshape(NW, NCH_D, CH)
    dst_d = edge_index[1].reshape(NW, NCH_D, CH)
    src_a = edge_index[0].reshape(NS, NCH_A, CH)
    dst_a = edge_index[1].reshape(NS, NCH_A, CH)
    zn = jnp.zeros((NPAD,), jnp.float32)
    dpart, w = _denom(s8, src_d, dst_d, zn)
    w_a = w.reshape(NS, NCH_A, CH)
    accs = _agg(h2, w_a, src_a, dst_a, dpart)
    return _elu(accs)[:N]


# speedup vs baseline: 30.2273x; 1.0489x over previous
import functools

import jax
import jax.numpy as jnp
from jax import lax
from jax.experimental import pallas as pl
from jax.experimental.pallas import tpu as pltpu
from jax.experimental.pallas import tpu_sc as plsc

N = 10000
E = 320000
D = 128
DH = D // 2
ALPHA = 0.2

NC = 2
NS = 16
L = 16
NW = NC * NS
EPW = E // NW
EPT = E // NS
CH = 80
NCH_D = EPW // CH
NCH_A = EPT // CH
NPAD = 10240
RPT = NPAD // NS

_mesh = plsc.VectorSubcoreMesh(core_axis_name="c", subcore_axis_name="s")
_sc_params = pltpu.CompilerParams(
    needs_layout_passes=False, use_tc_tiling_on_sc=False)


def _proj_body(x_ref, w0_ref, w1_ref, a80_ref, a81_ref, h2_ref, s8_ref):
    x = x_ref[...]
    h0 = jnp.dot(x, w0_ref[...], preferred_element_type=jnp.float32)
    h1 = jnp.dot(x, w1_ref[...], preferred_element_type=jnp.float32)
    h2_ref[0] = h0
    h2_ref[1] = h1
    s8_ref[...] = (
        lax.dot_general(a80_ref[...], h0, (((1,), (1,)), ((), ())),
                        preferred_element_type=jnp.float32)
        + lax.dot_general(a81_ref[...], h1, (((1,), (1,)), ((), ())),
                          preferred_element_type=jnp.float32))


_proj = pl.pallas_call(
    _proj_body,
    out_shape=[
        jax.ShapeDtypeStruct((NC, N, DH), jnp.float32),
        jax.ShapeDtypeStruct((8, N), jnp.float32),
    ],
)


def _denom_body(s8, src2, dst2, zn, dpart, w_hbm,
                s1_v, s2_v, si_v, di_v, w_v, dacc, dsem):
    c = lax.axis_index("c")
    s = lax.axis_index("s")
    wid = c * NS + s
    pltpu.sync_copy(s8.at[0], s1_v)
    pltpu.sync_copy(s8.at[1], s2_v)
    pltpu.sync_copy(src2.at[wid], si_v)
    pltpu.sync_copy(dst2.at[wid], di_v)
    pltpu.sync_copy(zn.at[pl.ds(s * RPT, RPT)], dacc.at[pl.ds(s * RPT, RPT)])

    @pl.loop(0, NCH_D)
    def _compute(j):
        for k in range(CH // L):
            si = si_v[j, pl.ds(k * L, L)]
            di = di_v[j, pl.ds(k * L, L)]
            e = plsc.load_gather(s1_v, [si]) + plsc.load_gather(s2_v, [di])
            e = jnp.where(e > 0.0, e, ALPHA * e)
            w_v[j, pl.ds(k * L, L)] = jnp.exp(e)

    pltpu.sync_copy(w_v, w_hbm.at[wid])
    plsc.subcore_barrier()

    @pl.loop(0, NCH_D)
    def _scatter(j):
        pltpu.async_copy(w_v.at[j], dacc.at[si_v.at[j]], dsem, add=True)

        @pl.when(j >= 8)
        def _throttle():
            pltpu.make_async_copy(w_v.at[0], dacc.at[si_v.at[0]], dsem).wait()

    @pl.loop(0, 8)
    def _drain(j):
        pltpu.make_async_copy(w_v.at[0], dacc.at[si_v.at[0]], dsem).wait()

    plsc.subcore_barrier()
    pltpu.sync_copy(dacc.at[pl.ds(s * RPT, RPT)],
                    dpart.at[c, pl.ds(s * RPT, RPT)])


_denom = functools.partial(
    pl.kernel,
    out_type=[
        jax.ShapeDtypeStruct((NC, NPAD), jnp.float32),
        jax.ShapeDtypeStruct((NW, NCH_D, CH), jnp.float32),
    ],
    mesh=_mesh,
    scratch_types=[
        pltpu.VMEM((N,), jnp.float32),
        pltpu.VMEM((N,), jnp.float32),
        pltpu.VMEM((NCH_D, CH), jnp.int32),
        pltpu.VMEM((NCH_D, CH), jnp.int32),
        pltpu.VMEM((NCH_D, CH), jnp.float32),
        pltpu.VMEM_SHARED((NPAD,), jnp.float32),
        pltpu.SemaphoreType.DMA,
    ],
    compiler_params=_sc_params,
)(_denom_body)


NB = 5
NQ = NCH_A // NB


def _agg_body(h2, w3, src2, dst2, dpart, out,
              d_v, d1_v, si_v, di_v,
              r0, r1, r2, r3, r4, w0, w1, w2, w3_, w4,
              g0, g1, g2, g3, g4, t0, t1, t2, t3, t4, acc):
    rows = (r0, r1, r2, r3, r4)
    wring = (w0, w1, w2, w3_, w4)
    gsem = (g0, g1, g2, g3, g4)
    ssem = (t0, t1, t2, t3, t4)
    c = lax.axis_index("c")
    s = lax.axis_index("s")
    pltpu.sync_copy(dpart.at[0], d_v)
    pltpu.sync_copy(dpart.at[1], d1_v)
    pltpu.sync_copy(src2.at[s], si_v)
    pltpu.sync_copy(dst2.at[s], di_v)

    @pl.loop(0, CH)
    def _zrow(r):
        for cb in range(DH // L):
            r0[r, pl.ds(cb * L, L)] = jnp.zeros((L,), jnp.float32)

    for p in range(RPT // CH):
        pltpu.sync_copy(r0, acc.at[pl.ds(s * RPT + p * CH, CH)])

    @pl.loop(0, NPAD // L)
    def _sum_denoms(i):
        d_v[pl.ds(i * L, L)] = d_v[pl.ds(i * L, L)] + d1_v[pl.ds(i * L, L)]

    def _gather(j, b):
        pltpu.async_copy(h2.at[c].at[si_v.at[j]], rows[b], gsem[b])
        pltpu.async_copy(w3.at[s].at[j], wring[b], gsem[b])

    def _wait_gather(b):
        pltpu.make_async_copy(h2.at[c].at[si_v.at[0]], rows[b], gsem[b]).wait()
        pltpu.make_async_copy(w3.at[s].at[0], wring[b], gsem[b]).wait()

    def _scatter(j, b):
        pltpu.async_copy(rows[b], acc.at[di_v.at[j]], ssem[b], add=True)

    def _wait_scatter(b):
        pltpu.make_async_copy(rows[b], acc.at[di_v.at[0]], ssem[b]).wait()

    def _compute(j, b):
        rv = rows[b]
        wv = wring[b]
        for k in range(CH // L):
            si = si_v[j, pl.ds(k * L, L)]
            att16 = wv[pl.ds(k * L, L)] / plsc.load_gather(d_v, [si])
            for t in range(L):
                a = att16[t]
                r = k * L + t
                for cb in range(DH // L):
                    rv[r, pl.ds(cb * L, L)] = rv[r, pl.ds(cb * L, L)] * a

    plsc.subcore_barrier()

    _gather(0, 0)
    _gather(1, 1)
    _gather(2, 2)

    @pl.loop(0, NQ)
    def _pipe(q):
        for i in range(NB):
            j = q * NB + i
            b3 = (i + 3) % NB
            jn = j + 3
            _wait_gather(i)

            @pl.when(jnp.logical_and(jn >= NB, jn < NCH_A))
            def _():
                _wait_scatter(b3)

            @pl.when(jn < NCH_A)
            def _():
                _gather(jn, b3)

            _compute(j, i)
            _scatter(j, i)

    for b in range(NB):
        _wait_scatter(b)

    plsc.subcore_barrier()

    row_base = s * RPT
    for p in range(RPT // CH):
        b = p % 2
        if p >= 2:
            pltpu.make_async_copy(
                rows[b], out.at[pl.ds(0, CH), pl.ds(0, DH)], ssem[b]).wait()
        pltpu.sync_copy(acc.at[pl.ds(row_base + p * CH, CH)], rows[b])
        rv = rows[b]

        @pl.loop(0, CH)
        def _elu_row(r):
            for cb in range(DH // L):
                y = rv[r, pl.ds(cb * L, L)]
                rv[r, pl.ds(cb * L, L)] = jnp.where(
                    y > 0.0, y, jnp.exp(y) - 1.0)

        pltpu.async_copy(
            rows[b],
            out.at[pl.ds(row_base + p * CH, CH), pl.ds(c * DH, DH)],
            ssem[b])
    for b in range(2):
        pltpu.make_async_copy(
            rows[b], out.at[pl.ds(0, CH), pl.ds(0, DH)], ssem[b]).wait()


_agg = functools.partial(
    pl.kernel,
    out_type=jax.ShapeDtypeStruct((NPAD, D), jnp.float32),
    mesh=_mesh,
    scratch_types=[
        pltpu.VMEM((NPAD,), jnp.float32),
        pltpu.VMEM((NPAD,), jnp.float32),
        pltpu.VMEM((NCH_A, CH), jnp.int32),
        pltpu.VMEM((NCH_A, CH), jnp.int32),
    ] + [pltpu.VMEM((CH, DH), jnp.float32) for _ in range(NB)]
    + [pltpu.VMEM((CH,), jnp.float32) for _ in range(NB)]
    + [pltpu.SemaphoreType.DMA for _ in range(2 * NB)]
    + [
        pltpu.VMEM_SHARED((NPAD, DH), jnp.float32),
    ],
    compiler_params=_sc_params,
)(_agg_body)


def kernel(x, edge_index, W, a):
    a1 = a[:D]
    a2 = a[D:]
    a80 = jnp.zeros((8, DH), jnp.float32).at[0].set(a1[:DH]).at[1].set(a2[:DH])
    a81 = jnp.zeros((8, DH), jnp.float32).at[0].set(a1[DH:]).at[1].set(a2[DH:])
    h2, s8 = _proj(x, W[:, :DH], W[:, DH:], a80, a81)
    src_d = edge_index[0].reshape(NW, NCH_D, CH)
    dst_d = edge_index[1].reshape(NW, NCH_D, CH)
    src_a = edge_index[0].reshape(NS, NCH_A, CH)
    dst_a = edge_index[1].reshape(NS, NCH_A, CH)
    zn = jnp.zeros((NPAD,), jnp.float32)
    dpart, w = _denom(s8, src_d, dst_d, zn)
    w_a = w.reshape(NS, NCH_A, CH)
    return _agg(h2, w_a, src_a, dst_a, dpart)[:N]


# speedup vs baseline: 30.9332x; 1.0234x over previous
import functools

import jax
import jax.numpy as jnp
from jax import lax
from jax.experimental import pallas as pl
from jax.experimental.pallas import tpu as pltpu
from jax.experimental.pallas import tpu_sc as plsc

N = 10000
E = 320000
D = 128
DH = D // 2
ALPHA = 0.2

NC = 2
NS = 16
L = 16
NW = NC * NS
EPW = E // NW
EPT = E // NS
CH = 80
NCH_D = EPW // CH
NCH_A = EPT // CH
NPAD = 10240
RPT = NPAD // NS

_mesh = plsc.VectorSubcoreMesh(core_axis_name="c", subcore_axis_name="s")
_sc_params = pltpu.CompilerParams(
    needs_layout_passes=False, use_tc_tiling_on_sc=False)


def _proj_s8_body(x_ref, w_ref, a8_ref, s8_ref):
    a8w = lax.dot_general(a8_ref[...], w_ref[...], (((1,), (1,)), ((), ())),
                          preferred_element_type=jnp.float32)
    s8_ref[...] = lax.dot_general(a8w, x_ref[...], (((1,), (1,)), ((), ())),
                                  preferred_element_type=jnp.float32)


_proj_s8 = pl.pallas_call(
    _proj_s8_body,
    out_shape=jax.ShapeDtypeStruct((8, N), jnp.float32),
)

_BRH = 1000


def _proj_h_body(x_ref, w0_ref, w1_ref, h2_ref):
    x = x_ref[...]
    h2_ref[0] = jnp.dot(x, w0_ref[...], preferred_element_type=jnp.float32)
    h2_ref[1] = jnp.dot(x, w1_ref[...], preferred_element_type=jnp.float32)


_proj_h = pl.pallas_call(
    _proj_h_body,
    grid=(N // _BRH,),
    in_specs=[
        pl.BlockSpec((_BRH, D), lambda i: (i, 0)),
        pl.BlockSpec((D, DH), lambda i: (0, 0)),
        pl.BlockSpec((D, DH), lambda i: (0, 0)),
    ],
    out_specs=pl.BlockSpec((NC, _BRH, DH), lambda i: (0, i, 0)),
    out_shape=jax.ShapeDtypeStruct((NC, N, DH), jnp.float32),
)


def _denom_body(s8, src2, dst2, zn, dpart, w_hbm,
                s1_v, s2_v, si_v, di_v, w_v, dacc, dsem):
    c = lax.axis_index("c")
    s = lax.axis_index("s")
    wid = c * NS + s
    pltpu.sync_copy(s8.at[0], s1_v)
    pltpu.sync_copy(s8.at[1], s2_v)
    pltpu.sync_copy(src2.at[wid], si_v)
    pltpu.sync_copy(dst2.at[wid], di_v)
    pltpu.sync_copy(zn.at[pl.ds(s * RPT, RPT)], dacc.at[pl.ds(s * RPT, RPT)])

    @pl.loop(0, NCH_D)
    def _compute(j):
        for k in range(CH // L):
            si = si_v[j, pl.ds(k * L, L)]
            di = di_v[j, pl.ds(k * L, L)]
            e = plsc.load_gather(s1_v, [si]) + plsc.load_gather(s2_v, [di])
            e = jnp.where(e > 0.0, e, ALPHA * e)
            w_v[j, pl.ds(k * L, L)] = jnp.exp(e)

    pltpu.sync_copy(w_v, w_hbm.at[wid])
    plsc.subcore_barrier()

    @pl.loop(0, NCH_D)
    def _scatter(j):
        pltpu.async_copy(w_v.at[j], dacc.at[si_v.at[j]], dsem, add=True)

        @pl.when(j >= 8)
        def _throttle():
            pltpu.make_async_copy(w_v.at[0], dacc.at[si_v.at[0]], dsem).wait()

    @pl.loop(0, 8)
    def _drain(j):
        pltpu.make_async_copy(w_v.at[0], dacc.at[si_v.at[0]], dsem).wait()

    plsc.subcore_barrier()
    pltpu.sync_copy(dacc.at[pl.ds(s * RPT, RPT)],
                    dpart.at[c, pl.ds(s * RPT, RPT)])


_denom = functools.partial(
    pl.kernel,
    out_type=[
        jax.ShapeDtypeStruct((NC, NPAD), jnp.float32),
        jax.ShapeDtypeStruct((NW, NCH_D, CH), jnp.float32),
    ],
    mesh=_mesh,
    scratch_types=[
        pltpu.VMEM((N,), jnp.float32),
        pltpu.VMEM((N,), jnp.float32),
        pltpu.VMEM((NCH_D, CH), jnp.int32),
        pltpu.VMEM((NCH_D, CH), jnp.int32),
        pltpu.VMEM((NCH_D, CH), jnp.float32),
        pltpu.VMEM_SHARED((NPAD,), jnp.float32),
        pltpu.SemaphoreType.DMA,
    ],
    compiler_params=_sc_params,
)(_denom_body)


NB = 5
NQ = NCH_A // NB


def _agg_body(h2, w3, src2, dst2, dpart, out,
              d_v, d1_v, si_v, di_v,
              r0, r1, r2, r3, r4, w0, w1, w2, w3_, w4,
              g0, g1, g2, g3, g4, t0, t1, t2, t3, t4, acc):
    rows = (r0, r1, r2, r3, r4)
    wring = (w0, w1, w2, w3_, w4)
    gsem = (g0, g1, g2, g3, g4)
    ssem = (t0, t1, t2, t3, t4)
    c = lax.axis_index("c")
    s = lax.axis_index("s")
    pltpu.sync_copy(dpart.at[0], d_v)
    pltpu.sync_copy(dpart.at[1], d1_v)
    pltpu.sync_copy(src2.at[s], si_v)
    pltpu.sync_copy(dst2.at[s], di_v)

    @pl.loop(0, CH)
    def _zrow(r):
        for cb in range(DH // L):
            r0[r, pl.ds(cb * L, L)] = jnp.zeros((L,), jnp.float32)

    for p in range(RPT // CH):
        pltpu.sync_copy(r0, acc.at[pl.ds(s * RPT + p * CH, CH)])

    @pl.loop(0, NPAD // L)
    def _sum_denoms(i):
        d_v[pl.ds(i * L, L)] = d_v[pl.ds(i * L, L)] + d1_v[pl.ds(i * L, L)]

    def _gather(j, b):
        pltpu.async_copy(h2.at[c].at[si_v.at[j]], rows[b], gsem[b])
        pltpu.async_copy(w3.at[s].at[j], wring[b], gsem[b])

    def _wait_gather(b):
        pltpu.make_async_copy(h2.at[c].at[si_v.at[0]], rows[b], gsem[b]).wait()
        pltpu.make_async_copy(w3.at[s].at[0], wring[b], gsem[b]).wait()

    def _scatter(j, b):
        pltpu.async_copy(rows[b], acc.at[di_v.at[j]], ssem[b], add=True)

    def _wait_scatter(b):
        pltpu.make_async_copy(rows[b], acc.at[di_v.at[0]], ssem[b]).wait()

    def _compute(j, b):
        rv = rows[b]
        wv = wring[b]
        for k in range(CH // L):
            si = si_v[j, pl.ds(k * L, L)]
            att16 = wv[pl.ds(k * L, L)] / plsc.load_gather(d_v, [si])
            for t in range(L):
                a = att16[t]
                r = k * L + t
                for cb in range(DH // L):
                    rv[r, pl.ds(cb * L, L)] = rv[r, pl.ds(cb * L, L)] * a

    plsc.subcore_barrier()

    _gather(0, 0)
    _gather(1, 1)
    _gather(2, 2)

    @pl.loop(0, NQ)
    def _pipe(q):
        for i in range(NB):
            j = q * NB + i
            b3 = (i + 3) % NB
            jn = j + 3
            _wait_gather(i)

            @pl.when(jnp.logical_and(jn >= NB, jn < NCH_A))
            def _():
                _wait_scatter(b3)

            @pl.when(jn < NCH_A)
            def _():
                _gather(jn, b3)

            _compute(j, i)
            _scatter(j, i)

    for b in range(NB):
        _wait_scatter(b)

    plsc.subcore_barrier()

    row_base = s * RPT
    for p in range(RPT // CH):
        b = p % 2
        if p >= 2:
            pltpu.make_async_copy(
                rows[b], out.at[pl.ds(0, CH), pl.ds(0, DH)], ssem[b]).wait()
        pltpu.sync_copy(acc.at[pl.ds(row_base + p * CH, CH)], rows[b])
        rv = rows[b]

        @pl.loop(0, CH)
        def _elu_row(r):
            for cb in range(DH // L):
                y = rv[r, pl.ds(cb * L, L)]
                rv[r, pl.ds(cb * L, L)] = jnp.where(
                    y > 0.0, y, jnp.exp(y) - 1.0)

        pltpu.async_copy(
            rows[b],
            out.at[pl.ds(row_base + p * CH, CH), pl.ds(c * DH, DH)],
            ssem[b])
    for b in range(2):
        pltpu.make_async_copy(
            rows[b], out.at[pl.ds(0, CH), pl.ds(0, DH)], ssem[b]).wait()


_agg = functools.partial(
    pl.kernel,
    out_type=jax.ShapeDtypeStruct((NPAD, D), jnp.float32),
    mesh=_mesh,
    scratch_types=[
        pltpu.VMEM((NPAD,), jnp.float32),
        pltpu.VMEM((NPAD,), jnp.float32),
        pltpu.VMEM((NCH_A, CH), jnp.int32),
        pltpu.VMEM((NCH_A, CH), jnp.int32),
    ] + [pltpu.VMEM((CH, DH), jnp.float32) for _ in range(NB)]
    + [pltpu.VMEM((CH,), jnp.float32) for _ in range(NB)]
    + [pltpu.SemaphoreType.DMA for _ in range(2 * NB)]
    + [
        pltpu.VMEM_SHARED((NPAD, DH), jnp.float32),
    ],
    compiler_params=_sc_params,
)(_agg_body)


def kernel(x, edge_index, W, a):
    a8 = jnp.zeros((8, D), jnp.float32).at[0].set(a[:D]).at[1].set(a[D:])
    s8 = _proj_s8(x, W, a8)
    h2 = _proj_h(x, W[:, :DH], W[:, DH:])
    src_d = edge_index[0].reshape(NW, NCH_D, CH)
    dst_d = edge_index[1].reshape(NW, NCH_D, CH)
    src_a = edge_index[0].reshape(NS, NCH_A, CH)
    dst_a = edge_index[1].reshape(NS, NCH_A, CH)
    zn = jnp.zeros((NPAD,), jnp.float32)
    dpart, w = _denom(s8, src_d, dst_d, zn)
    w_a = w.reshape(NS, NCH_A, CH)
    return _agg(h2, w_a, src_a, dst_a, dpart)[:N]
